# baseline (speedup 1.0000x reference)
import functools

import jax
import jax.numpy as jnp
from jax import lax
from jax.experimental import pallas as pl
from jax.experimental.pallas import tpu as pltpu
from jax.experimental.pallas import tpu_sc as plsc

_N = 64
_NT = 8
_PPT = _N // _NT

_mesh = plsc.VectorSubcoreMesh(
    core_axis_name="c", subcore_axis_name="s", num_cores=1, num_subcores=_NT
)


@functools.partial(
    pl.kernel,
    out_type=jax.ShapeDtypeStruct((_N,), jnp.float32),
    mesh=_mesh,
    scratch_types=[
        pltpu.VMEM((_PPT * 8,), jnp.float32),
        pltpu.VMEM((16,), jnp.float32),
        pltpu.SemaphoreType.DMA,
    ],
)
def _probe_gather(x_hbm, out_hbm, rows_v, out_v, sem):
    wid = lax.axis_index("s")

    @pl.when(wid < _NT)
    def _():
        copies = []
        for j in range(_PPT):
            y = 16 * _PPT * wid + 16 * j + 8
            c = 32 * _PPT * wid + 32 * j
            copies.append(
                pltpu.async_copy(
                    x_hbm.at[j, y, pl.ds(c, 8)], rows_v.at[pl.ds(8 * j, 8)], sem
                )
            )
        for cp in copies:
            cp.wait()
        lane = lax.iota(jnp.int32, 16)
        vals = jnp.zeros((16,), jnp.float32)
        for j in range(_PPT):
            v = rows_v[pl.ds(16 * (j // 2), 16)]
            vals = jnp.where(lane == j, v[8 * (j % 2)], vals)
        out_v[...] = vals
        pltpu.sync_copy(
            out_v.at[pl.ds(0, _PPT)], out_hbm.at[pl.ds(_PPT * wid, _PPT)]
        )


def kernel(x):
    return _probe_gather(x)
